# baseline (speedup 1.0000x reference)
import jax
import jax.numpy as jnp
from jax import lax
from jax.experimental import pallas as pl
from jax.experimental.pallas import tpu as pltpu
from jax.experimental.pallas import tpu_sc as plsc

N = 10000
E = 320000
D = 128
DH = D // 2

NC = 2
NS = 16
NW = NC * NS

CH = 128
NCH = 160
NBUF = 4
EPT = NCH * CH
PAD_E = NS * EPT

DUMMY_ROW = N
ACC_ROWS = 10008
ZROWS = 632

MBLK = 1000


def _sc_body(h_ref, src_ref, dst_ref, zero_ref, out_ref,
             src_v, dst_v, buf_a, buf_b, acc, sem_a, sem_b):
    c = lax.axis_index("c")
    s = lax.axis_index("s")

    pltpu.sync_copy(src_ref.at[s], src_v)
    pltpu.sync_copy(dst_ref.at[s], dst_v)

    @pl.when(s < NS - 1)
    def _():
        pltpu.sync_copy(zero_ref.at[pl.ds(0, ZROWS)],
                        acc.at[pl.ds(s * ZROWS, ZROWS)])

    @pl.when(s == NS - 1)
    def _():
        rem = ACC_ROWS - (NS - 1) * ZROWS
        pltpu.sync_copy(zero_ref.at[pl.ds(0, rem)],
                        acc.at[pl.ds((NS - 1) * ZROWS, rem)])

    plsc.subcore_barrier()

    h_c = h_ref.at[c]
    pltpu.async_copy(h_c.at[src_v.at[0]], buf_a, sem_a)

    def body(i, carry):
        j0 = 2 * i
        pltpu.async_copy(h_c.at[src_v.at[j0 + 1]], buf_b, sem_b)
        pltpu.make_async_copy(h_c.at[src_v.at[j0]], buf_a, sem_a).wait()
        pltpu.sync_copy(buf_a, acc.at[dst_v.at[j0]], add=True)
        pltpu.async_copy(h_c.at[src_v.at[j0 + 2]], buf_a, sem_a)
        pltpu.make_async_copy(h_c.at[src_v.at[j0 + 1]], buf_b, sem_b).wait()
        pltpu.sync_copy(buf_b, acc.at[dst_v.at[j0 + 1]], add=True)
        return carry

    lax.fori_loop(0, NCH // 2, body, 0)
    pltpu.make_async_copy(h_c.at[src_v.at[NCH]], buf_a, sem_a).wait()
    plsc.subcore_barrier()

    @pl.when(s < NS - 1)
    def _():
        pltpu.sync_copy(acc.at[pl.ds(s * ZROWS, ZROWS)],
                        out_ref.at[c, pl.ds(s * ZROWS, ZROWS)])

    @pl.when(s == NS - 1)
    def _():
        rem = N - (NS - 1) * ZROWS
        pltpu.sync_copy(acc.at[pl.ds((NS - 1) * ZROWS, rem)],
                        out_ref.at[c, pl.ds((NS - 1) * ZROWS, rem)])


_sc_scatter = pl.kernel(
    _sc_body,
    out_type=jax.ShapeDtypeStruct((NC, N, DH), jnp.float32),
    mesh=plsc.VectorSubcoreMesh(core_axis_name="c", subcore_axis_name="s",
                                num_cores=NC, num_subcores=NS),
    scratch_types=[
        pltpu.VMEM((NCH + 1, CH), jnp.int32),
        pltpu.VMEM((NCH, CH), jnp.int32),
        pltpu.VMEM((CH, DH), jnp.float32),
        pltpu.VMEM((CH, DH), jnp.float32),
        pltpu.VMEM_SHARED((ACC_ROWS, DH), jnp.float32),
        pltpu.SemaphoreType.DMA,
        pltpu.SemaphoreType.DMA,
    ],
    compiler_params=pltpu.CompilerParams(use_tc_tiling_on_sc=False),
)


def _split_store(o_ref, r):
    o_ref[0] = r[:, :DH]
    o_ref[1] = r[:, DH:]


def _mm_body(x_ref, w_ref, o_ref):
    r = jnp.dot(x_ref[...], w_ref[...], preferred_element_type=jnp.float32)
    _split_store(o_ref, r)


def _act_mm_body(agg_ref, b_ref, w_ref, o_ref):
    a = jnp.concatenate([agg_ref[0], agg_ref[1]], axis=-1)
    h = jnp.maximum(a + b_ref[...], 0.0)
    r = jnp.dot(h, w_ref[...], preferred_element_type=jnp.float32)
    _split_store(o_ref, r)


def _sig_body(agg_ref, b_ref, o_ref):
    a = jnp.concatenate([agg_ref[0], agg_ref[1]], axis=-1)
    o_ref[...] = jax.nn.sigmoid(a + b_ref[...])


_mm = pl.pallas_call(
    _mm_body,
    grid=(N // MBLK,),
    in_specs=[
        pl.BlockSpec((MBLK, D), lambda i: (i, 0)),
        pl.BlockSpec((D, D), lambda i: (0, 0)),
    ],
    out_specs=pl.BlockSpec((NC, MBLK, DH), lambda i: (0, i, 0)),
    out_shape=jax.ShapeDtypeStruct((NC, N, DH), jnp.float32),
)

_act_mm = pl.pallas_call(
    _act_mm_body,
    grid=(N // MBLK,),
    in_specs=[
        pl.BlockSpec((NC, MBLK, DH), lambda i: (0, i, 0)),
        pl.BlockSpec((1, D), lambda i: (0, 0)),
        pl.BlockSpec((D, D), lambda i: (0, 0)),
    ],
    out_specs=pl.BlockSpec((NC, MBLK, DH), lambda i: (0, i, 0)),
    out_shape=jax.ShapeDtypeStruct((NC, N, DH), jnp.float32),
)

_sig = pl.pallas_call(
    _sig_body,
    grid=(N // MBLK,),
    in_specs=[
        pl.BlockSpec((NC, MBLK, DH), lambda i: (0, i, 0)),
        pl.BlockSpec((1, D), lambda i: (0, 0)),
    ],
    out_specs=pl.BlockSpec((MBLK, D), lambda i: (i, 0)),
    out_shape=jax.ShapeDtypeStruct((N, D), jnp.float32),
)


def kernel(x, edge_index, W1, b1, W2, b2, W3, b3):
    src = edge_index[0].astype(jnp.int32)
    dst = edge_index[1].astype(jnp.int32)
    src_p = jnp.concatenate(
        [src, jnp.zeros((PAD_E - E,), jnp.int32)]).reshape(NS, NCH, CH)
    src_p = jnp.concatenate(
        [src_p, jnp.zeros((NS, 1, CH), jnp.int32)], axis=1)
    dst_p = jnp.concatenate(
        [dst, jnp.full((PAD_E - E,), DUMMY_ROW, jnp.int32)]).reshape(NS, NCH, CH)
    zeros = jnp.zeros((ZROWS, DH), jnp.float32)

    b1r = b1.reshape(1, D)
    b2r = b2.reshape(1, D)
    b3r = b3.reshape(1, D)

    t = _mm(x, W1)
    agg = _sc_scatter(t, src_p, dst_p, zeros)
    t = _act_mm(agg, b1r, W2)
    agg = _sc_scatter(t, src_p, dst_p, zeros)
    t = _act_mm(agg, b2r, W3)
    agg = _sc_scatter(t, src_p, dst_p, zeros)
    return _sig(agg, b3r)


# speedup vs baseline: 1.4875x; 1.4875x over previous
import jax
import jax.numpy as jnp
from jax import lax
from jax.experimental import pallas as pl
from jax.experimental.pallas import tpu as pltpu
from jax.experimental.pallas import tpu_sc as plsc

N = 10000
E = 320000
D = 128
DH = D // 2

NC = 2
NS = 16
NW = NC * NS

CH = 128
NCH = 158
EPT = NCH * CH
PAD_E = NS * EPT

DUMMY_ROW = N
ACC_ROWS = 10008
ZROWS = 632

MBLK = 1000


def _sc_body(h_ref, src_ref, dst_ref, zero_ref, out_ref,
             src_v, dst_v, buf_a, buf_b, acc, sem_a, sem_b):
    c = lax.axis_index("c")
    s = lax.axis_index("s")

    pltpu.sync_copy(src_ref.at[s], src_v)
    pltpu.sync_copy(dst_ref.at[s], dst_v)

    @pl.when(s < NS - 1)
    def _():
        pltpu.sync_copy(zero_ref.at[pl.ds(0, ZROWS)],
                        acc.at[pl.ds(s * ZROWS, ZROWS)])

    @pl.when(s == NS - 1)
    def _():
        rem = ACC_ROWS - (NS - 1) * ZROWS
        pltpu.sync_copy(zero_ref.at[pl.ds(0, rem)],
                        acc.at[pl.ds((NS - 1) * ZROWS, rem)])

    plsc.subcore_barrier()

    h_c = h_ref.at[c]
    pltpu.async_copy(h_c.at[src_v.at[0]], buf_a, sem_a)

    def body(i, carry):
        j0 = 2 * i
        pltpu.async_copy(h_c.at[src_v.at[j0 + 1]], buf_b, sem_b)
        pltpu.make_async_copy(h_c.at[src_v.at[j0]], buf_a, sem_a).wait()
        pltpu.sync_copy(buf_a, acc.at[dst_v.at[j0]], add=True)

        @pl.when(i < NCH // 2 - 1)
        def _():
            pltpu.async_copy(h_c.at[src_v.at[j0 + 2]], buf_a, sem_a)

        pltpu.make_async_copy(h_c.at[src_v.at[j0 + 1]], buf_b, sem_b).wait()
        pltpu.sync_copy(buf_b, acc.at[dst_v.at[j0 + 1]], add=True)
        return carry

    lax.fori_loop(0, NCH // 2, body, 0)
    plsc.subcore_barrier()

    @pl.when(s < NS - 1)
    def _():
        pltpu.sync_copy(acc.at[pl.ds(s * ZROWS, ZROWS)],
                        out_ref.at[c, pl.ds(s * ZROWS, ZROWS)])

    @pl.when(s == NS - 1)
    def _():
        rem = N - (NS - 1) * ZROWS
        pltpu.sync_copy(acc.at[pl.ds((NS - 1) * ZROWS, rem)],
                        out_ref.at[c, pl.ds((NS - 1) * ZROWS, rem)])


_sc_scatter = pl.kernel(
    _sc_body,
    out_type=jax.ShapeDtypeStruct((NC, N, DH), jnp.float32),
    mesh=plsc.VectorSubcoreMesh(core_axis_name="c", subcore_axis_name="s",
                                num_cores=NC, num_subcores=NS),
    scratch_types=[
        pltpu.VMEM((NCH, CH), jnp.int32),
        pltpu.VMEM((NCH, CH), jnp.int32),
        pltpu.VMEM((CH, DH), jnp.float32),
        pltpu.VMEM((CH, DH), jnp.float32),
        pltpu.VMEM_SHARED((ACC_ROWS, DH), jnp.float32),
        pltpu.SemaphoreType.DMA,
        pltpu.SemaphoreType.DMA,
    ],
    compiler_params=pltpu.CompilerParams(use_tc_tiling_on_sc=False),
)


def _split_store(o_ref, r):
    o_ref[0] = r[:, :DH]
    o_ref[1] = r[:, DH:]


def _mm_body(x_ref, w_ref, o_ref):
    r = jnp.dot(x_ref[...], w_ref[...], preferred_element_type=jnp.float32)
    _split_store(o_ref, r)


def _act_mm_body(agg_ref, b_ref, w_ref, o_ref):
    a = jnp.concatenate([agg_ref[0], agg_ref[1]], axis=-1)
    h = jnp.maximum(a + b_ref[...], 0.0)
    r = jnp.dot(h, w_ref[...], preferred_element_type=jnp.float32)
    _split_store(o_ref, r)


def _sig_body(agg_ref, b_ref, o_ref):
    a = jnp.concatenate([agg_ref[0], agg_ref[1]], axis=-1)
    o_ref[...] = jax.nn.sigmoid(a + b_ref[...])


_mm = pl.pallas_call(
    _mm_body,
    grid=(N // MBLK,),
    in_specs=[
        pl.BlockSpec((MBLK, D), lambda i: (i, 0)),
        pl.BlockSpec((D, D), lambda i: (0, 0)),
    ],
    out_specs=pl.BlockSpec((NC, MBLK, DH), lambda i: (0, i, 0)),
    out_shape=jax.ShapeDtypeStruct((NC, N, DH), jnp.float32),
)

_act_mm = pl.pallas_call(
    _act_mm_body,
    grid=(N // MBLK,),
    in_specs=[
        pl.BlockSpec((NC, MBLK, DH), lambda i: (0, i, 0)),
        pl.BlockSpec((1, D), lambda i: (0, 0)),
        pl.BlockSpec((D, D), lambda i: (0, 0)),
    ],
    out_specs=pl.BlockSpec((NC, MBLK, DH), lambda i: (0, i, 0)),
    out_shape=jax.ShapeDtypeStruct((NC, N, DH), jnp.float32),
)

_sig = pl.pallas_call(
    _sig_body,
    grid=(N // MBLK,),
    in_specs=[
        pl.BlockSpec((NC, MBLK, DH), lambda i: (0, i, 0)),
        pl.BlockSpec((1, D), lambda i: (0, 0)),
    ],
    out_specs=pl.BlockSpec((MBLK, D), lambda i: (i, 0)),
    out_shape=jax.ShapeDtypeStruct((N, D), jnp.float32),
)


def kernel(x, edge_index, W1, b1, W2, b2, W3, b3):
    src = edge_index[0].astype(jnp.int32)
    dst = edge_index[1].astype(jnp.int32)
    src_p = jnp.concatenate(
        [src, jnp.zeros((PAD_E - E,), jnp.int32)]).reshape(NS, NCH, CH)
    dst_p = jnp.concatenate(
        [dst, jnp.full((PAD_E - E,), DUMMY_ROW, jnp.int32)]).reshape(NS, NCH, CH)
    zeros = jnp.zeros((ZROWS, DH), jnp.float32)

    b1r = b1.reshape(1, D)
    b2r = b2.reshape(1, D)
    b3r = b3.reshape(1, D)

    t = _mm(x, W1)
    agg = _sc_scatter(t, src_p, dst_p, zeros)
    t = _act_mm(agg, b1r, W2)
    agg = _sc_scatter(t, src_p, dst_p, zeros)
    t = _act_mm(agg, b2r, W3)
    agg = _sc_scatter(t, src_p, dst_p, zeros)
    return _sig(agg, b3r)
